# Initial kernel scaffold; baseline (speedup 1.0000x reference)
import jax
import jax.numpy as jnp
from jax.experimental import pallas as pl


def kernel(hidden_states, hash_proj, W1, b1, W2, b2):
    raise NotImplementedError("write your pallas kernel here")


# speedup vs baseline: 2.8044x; 2.8044x over previous
import functools

import jax
import jax.numpy as jnp
from jax import lax
from jax.experimental import pallas as pl
from jax.experimental.pallas import tpu as pltpu
from jax.experimental.pallas import tpu_sc as plsc

D_MODEL = 1024
D_FF = 4096
E = 8
T = 2048
NP = 2 * T
BM = 128
L = NP + E * BM
NB = L // BM
NBP = ((NB + 15) // 16) * 16
FF2 = D_FF // 2

NW = 32


@functools.cache
def _mesh():
    return plsc.VectorSubcoreMesh(core_axis_name="c", subcore_axis_name="s",
                                  num_cores=2, num_subcores=16)


def _route_body(x_ref, p_ref, sel_ref):
    x = x_ref[...]
    lg = lax.dot_general(x, p_ref[...], (((1,), (0,)), ((), ())),
                         preferred_element_type=jnp.float32)
    l = lg[:, 0:8] + lg[:, 8:16] + lg[:, 16:24] + lg[:, 24:32]
    iota = lax.broadcasted_iota(jnp.int32, (T, E), 1)
    m1 = jnp.max(l, axis=1, keepdims=True)
    i1 = jnp.min(jnp.where(l == m1, iota, E), axis=1, keepdims=True)
    masked = jnp.where(iota == i1, -jnp.inf, l)
    m2 = jnp.max(masked, axis=1, keepdims=True)
    i2 = jnp.min(jnp.where(masked == m2, iota, E), axis=1, keepdims=True)
    sel_ref[:, 0:1] = i1
    sel_ref[:, 1:2] = i2


def _route(x, pmat):
    return pl.pallas_call(
        _route_body,
        out_shape=jax.ShapeDtypeStruct((T, 2), jnp.int32),
    )(x, pmat)


def _bookkeep_body(eid_hbm, slot_hbm, tok_hbm, be_hbm, bv_hbm,
                   eid_v, rank_v, slot_v, tok_v, starts_v, be_v, bv_v,
                   cnt_s, ends_s):
    wid = lax.axis_index("s") * 2 + lax.axis_index("c")

    @pl.when(wid == 0)
    def _():
        pltpu.sync_copy(eid_hbm, eid_v)
        for e in range(E):
            cnt_s[e] = 0

        i16 = lax.iota(jnp.int32, 16)
        zeros16 = jnp.zeros((16,), jnp.int32)

        def pass1(c, carry):
            v = eid_v[pl.ds(c * 16, 16)]
            r = zeros16
            for e in range(E):
                m = v == jnp.full((16,), e, jnp.int32)
                mi = m.astype(jnp.int32)
                cs = plsc.cumsum(mi)
                base = jnp.full((16,), cnt_s[e] - 1, jnp.int32)
                r = r + jnp.where(m, base + cs, zeros16)
                cnt_s[e] = cnt_s[e] + jnp.sum(mi)
            rank_v[pl.ds(c * 16, 16)] = r
            return carry

        lax.fori_loop(0, NP // 16, pass1, 0)

        v_st = zeros16
        acc = jnp.int32(0)
        for e in range(E):
            v_st = jnp.where(i16 == jnp.full((16,), e, jnp.int32),
                             jnp.full((16,), acc, jnp.int32), v_st)
            region = ((cnt_s[e] + BM - 1) >> 7) << 7
            acc = acc + region
            ends_s[e] = acc
        starts_v[...] = v_st

        def init_tok(c, carry):
            tok_v[pl.ds(c * 16, 16)] = zeros16
            return carry

        lax.fori_loop(0, L // 16, init_tok, 0)

        def pass2(c, carry):
            v = eid_v[pl.ds(c * 16, 16)]
            sv = plsc.load_gather(starts_v, [v])
            slotc = sv + rank_v[pl.ds(c * 16, 16)]
            slot_v[pl.ds(c * 16, 16)] = slotc
            tokids = (jnp.full((16,), c * 16, jnp.int32) + i16) >> 1
            plsc.store_scatter(tok_v, [slotc], tokids)
            return carry

        lax.fori_loop(0, NP // 16, pass2, 0)

        for cb in range(NBP // 16):
            jb = (jnp.full((16,), cb * 16, jnp.int32) + i16) << 7
            be = zeros16
            for e in range(E):
                ee = jnp.full((16,), ends_s[e], jnp.int32)
                be = be + (jb >= ee).astype(jnp.int32)
            be_v[pl.ds(cb * 16, 16)] = jnp.minimum(be, jnp.full((16,), E - 1, jnp.int32))
            bv_v[pl.ds(cb * 16, 16)] = (jb < jnp.full((16,), ends_s[E - 1], jnp.int32)).astype(jnp.int32)

        pltpu.sync_copy(slot_v, slot_hbm)
        pltpu.sync_copy(tok_v, tok_hbm)
        pltpu.sync_copy(be_v, be_hbm)
        pltpu.sync_copy(bv_v, bv_hbm)


def _bookkeep(eid):
    f = functools.partial(
        pl.kernel,
        out_type=(
            jax.ShapeDtypeStruct((NP,), jnp.int32),
            jax.ShapeDtypeStruct((L,), jnp.int32),
            jax.ShapeDtypeStruct((NBP,), jnp.int32),
            jax.ShapeDtypeStruct((NBP,), jnp.int32),
        ),
        mesh=_mesh(),
        scratch_types=[
            pltpu.VMEM((NP,), jnp.int32),
            pltpu.VMEM((NP,), jnp.int32),
            pltpu.VMEM((NP,), jnp.int32),
            pltpu.VMEM((L,), jnp.int32),
            pltpu.VMEM((16,), jnp.int32),
            pltpu.VMEM((NBP,), jnp.int32),
            pltpu.VMEM((NBP,), jnp.int32),
            pltpu.SMEM((E,), jnp.int32),
            pltpu.SMEM((E,), jnp.int32),
        ],
        compiler_params=pltpu.CompilerParams(needs_layout_passes=False),
    )
    return f(_bookkeep_body)(eid)


_RPW = L // NW
_GCH = _RPW // 2


def _gather_body(tok_hbm, x_hbm, xs_hbm, idx_v, rows_v, sem):
    wid = lax.axis_index("s") * 2 + lax.axis_index("c")
    for k in range(2):
        b = wid * _RPW + k * _GCH
        pltpu.sync_copy(tok_hbm.at[pl.ds(b, _GCH)], idx_v)
        pltpu.async_copy(x_hbm.at[idx_v], rows_v, sem).wait()
        pltpu.sync_copy(rows_v, xs_hbm.at[pl.ds(b, _GCH)])


def _gather_rows(tok, x):
    f = functools.partial(
        pl.kernel,
        out_type=jax.ShapeDtypeStruct((L, D_MODEL), jnp.float32),
        mesh=_mesh(),
        scratch_types=[
            pltpu.VMEM((_GCH,), jnp.int32),
            pltpu.VMEM((_GCH, D_MODEL), jnp.float32),
            pltpu.SemaphoreType.DMA,
        ],
        compiler_params=pltpu.CompilerParams(needs_layout_passes=False),
    )
    return f(_gather_body)(tok, x)


def _mlp_body(acc_in, be_ref, bv_ref, x_ref, w1_ref, b1_ref, w2_ref, b2_ref,
              *rest):
    if acc_in:
        yin_ref, o_ref = rest
    else:
        (o_ref,) = rest
    j = pl.program_id(0)

    @pl.when(bv_ref[j] == 1)
    def _():
        xb = x_ref[...].astype(jnp.bfloat16)
        h = lax.dot_general(xb, w1_ref[0], (((1,), (0,)), ((), ())),
                            preferred_element_type=jnp.float32)
        h = h + b1_ref[0]
        h = h * 0.5 * (1.0 + lax.erf(h * (2.0 ** -0.5)))
        y = lax.dot_general(h.astype(jnp.bfloat16), w2_ref[0],
                            (((1,), (0,)), ((), ())),
                            preferred_element_type=jnp.float32)
        if acc_in:
            o_ref[...] = yin_ref[...] + (y + b2_ref[0]) * 0.5
        else:
            o_ref[...] = y * 0.5


def _mlp_half(ff, xs, w1, b1, w2, b2, be, bv, ypart):
    acc_in = ff == 1
    body = functools.partial(_mlp_body, acc_in)
    in_specs = [
        pl.BlockSpec((BM, D_MODEL), lambda j, be, bv: (j, 0)),
        pl.BlockSpec((1, D_MODEL, FF2), lambda j, be, bv: (be[j], 0, ff)),
        pl.BlockSpec((1, 1, FF2), lambda j, be, bv: (be[j], 0, ff)),
        pl.BlockSpec((1, FF2, D_MODEL), lambda j, be, bv: (be[j], ff, 0)),
        pl.BlockSpec((1, 1, D_MODEL), lambda j, be, bv: (be[j], 0, 0)),
    ]
    args = [be, bv, xs, w1, b1.reshape(E, 1, D_FF), w2,
            b2.reshape(E, 1, D_MODEL)]
    if acc_in:
        in_specs.append(pl.BlockSpec((BM, D_MODEL), lambda j, be, bv: (j, 0)))
        args.append(ypart)
    grid_spec = pltpu.PrefetchScalarGridSpec(
        num_scalar_prefetch=2,
        grid=(NB,),
        in_specs=in_specs,
        out_specs=pl.BlockSpec((BM, D_MODEL), lambda j, be, bv: (j, 0)),
    )
    return pl.pallas_call(
        body,
        grid_spec=grid_spec,
        out_shape=jax.ShapeDtypeStruct((L, D_MODEL), jnp.float32),
        compiler_params=pltpu.CompilerParams(vmem_limit_bytes=60 * 1024 * 1024),
    )(*args)


_TPW = T // NW


def _combine_gather_body(i0_hbm, i1_hbm, ys_hbm, ga_hbm, gb_hbm,
                         ia_v, ib_v, a_v, b_v, sem):
    wid = lax.axis_index("s") * 2 + lax.axis_index("c")
    for k in range(2):
        b = wid * _TPW + k * (_TPW // 2)
        pltpu.sync_copy(i0_hbm.at[pl.ds(b, _TPW // 2)], ia_v)
        pltpu.sync_copy(i1_hbm.at[pl.ds(b, _TPW // 2)], ib_v)
        pltpu.async_copy(ys_hbm.at[ia_v], a_v, sem).wait()
        pltpu.async_copy(ys_hbm.at[ib_v], b_v, sem).wait()
        pltpu.sync_copy(a_v, ga_hbm.at[pl.ds(b, _TPW // 2)])
        pltpu.sync_copy(b_v, gb_hbm.at[pl.ds(b, _TPW // 2)])


def _combine_gather(i0, i1, ys):
    f = functools.partial(
        pl.kernel,
        out_type=(
            jax.ShapeDtypeStruct((T, D_MODEL), jnp.float32),
            jax.ShapeDtypeStruct((T, D_MODEL), jnp.float32),
        ),
        mesh=_mesh(),
        scratch_types=[
            pltpu.VMEM((_TPW // 2,), jnp.int32),
            pltpu.VMEM((_TPW // 2,), jnp.int32),
            pltpu.VMEM((_TPW // 2, D_MODEL), jnp.float32),
            pltpu.VMEM((_TPW // 2, D_MODEL), jnp.float32),
            pltpu.SemaphoreType.DMA,
        ],
        compiler_params=pltpu.CompilerParams(needs_layout_passes=False),
    )
    return f(_combine_gather_body)(i0, i1, ys)


def _add_body(a_ref, b_ref, o_ref):
    o_ref[...] = a_ref[...] + b_ref[...]


def _final_add(ga, gb):
    blk = 256
    return pl.pallas_call(
        _add_body,
        grid=(T // blk,),
        in_specs=[pl.BlockSpec((blk, D_MODEL), lambda i: (i, 0)),
                  pl.BlockSpec((blk, D_MODEL), lambda i: (i, 0))],
        out_specs=pl.BlockSpec((blk, D_MODEL), lambda i: (i, 0)),
        out_shape=jax.ShapeDtypeStruct((T, D_MODEL), jnp.float32),
    )(ga, gb)


def kernel(hidden_states, hash_proj, W1, b1, W2, b2):
    orig_shape = hidden_states.shape
    x = hidden_states.reshape(T, D_MODEL)
    pmat = hash_proj.transpose(1, 0, 2).reshape(D_MODEL, 4 * E)

    sel = _route(x, pmat)
    eid = sel.reshape(NP)
    slot, tok, be, bv = _bookkeep(eid)
    xs = _gather_rows(tok, x)
    ypart = _mlp_half(0, xs, W1, b1, W2, b2, be, bv, None)
    ys = _mlp_half(1, xs, W1, b1, W2, b2, be, bv, ypart)
    pos = slot.reshape(T, 2)
    ga, gb = _combine_gather(pos[:, 0], pos[:, 1], ys)
    out = _final_add(ga, gb)
    return out.reshape(orig_shape)
